# baseline (speedup 1.0000x reference)
import functools

import jax
import jax.numpy as jnp
from jax import lax
from jax.experimental import pallas as pl
from jax.experimental.pallas import tpu as pltpu
from jax.experimental.pallas import tpu_sc as plsc

_LANES = 16
_NUM_CORES = 2
_NUM_SUBCORES = 16
_NW = _NUM_CORES * _NUM_SUBCORES


def _densify2(w0, ii0, io0, w1, ii1, io1, din, hdim, dout):
    nnz0, nnz1 = w0.shape[0], w1.shape[0]
    max_nnz_pw = max(nnz0, nnz1) // _NW
    max_words_pw = max(din * hdim, hdim * dout) // _NW

    mesh = plsc.VectorSubcoreMesh(core_axis_name="c", subcore_axis_name="s")

    nnz0_pw, nnz1_pw = nnz0 // _NW, nnz1 // _NW
    rows0_pw, rows1_pw = din // _NW, hdim // _NW

    @functools.partial(
        pl.kernel,
        out_type=(
            jax.ShapeDtypeStruct((din, hdim // 2), jnp.int32),
            jax.ShapeDtypeStruct((hdim, dout // 2), jnp.int32),
        ),
        mesh=mesh,
        compiler_params=pltpu.CompilerParams(needs_layout_passes=False),
        scratch_types=[
            pltpu.VMEM((rows0_pw, hdim // 2), jnp.int32),
            pltpu.VMEM((rows1_pw, dout // 2), jnp.int32),
            pltpu.VMEM((nnz0_pw,), jnp.float32),
            pltpu.VMEM((nnz0_pw,), jnp.int32),
            pltpu.VMEM((nnz0_pw,), jnp.int32),
            pltpu.VMEM((nnz1_pw,), jnp.float32),
            pltpu.VMEM((nnz1_pw,), jnp.int32),
            pltpu.VMEM((nnz1_pw,), jnp.int32),
            pltpu.SemaphoreType.DMA,
            pltpu.SemaphoreType.DMA,
            pltpu.SemaphoreType.DMA,
        ],
    )
    def sc_densify(w0_hbm, ii0_hbm, io0_hbm, w1_hbm, ii1_hbm, io1_hbm,
                   out0_hbm, out1_hbm, strip0, strip1,
                   wv0, ii0v, io0v, wv1, ii1v, io1v,
                   sem_in, sem_o0, sem_o1):
        wid = lax.axis_index("s") * _NUM_CORES + lax.axis_index("c")
        n0, n1 = wid * nnz0_pw, wid * nnz1_pw

        loads = [
            pltpu.async_copy(w0_hbm.at[pl.ds(n0, nnz0_pw)], wv0, sem_in),
            pltpu.async_copy(ii0_hbm.at[pl.ds(n0, nnz0_pw)], ii0v, sem_in),
            pltpu.async_copy(io0_hbm.at[pl.ds(n0, nnz0_pw)], io0v, sem_in),
            pltpu.async_copy(w1_hbm.at[pl.ds(n1, nnz1_pw)], wv1, sem_in),
            pltpu.async_copy(ii1_hbm.at[pl.ds(n1, nnz1_pw)], ii1v, sem_in),
            pltpu.async_copy(io1_hbm.at[pl.ds(n1, nnz1_pw)], io1v, sem_in),
        ]

        zeros = jnp.zeros((_LANES,), jnp.int32)

        def zero_strip(strip, rows, ncols_w):
            @pl.loop(0, rows)
            def _(r):
                @pl.loop(0, ncols_w, step=_LANES * 8)
                def _(c):
                    for u in range(8):
                        strip[r, pl.ds(c + u * _LANES, _LANES)] = zeros

        zero_strip(strip0, rows0_pw, hdim // 2)
        zero_strip(strip1, rows1_pw, dout // 2)
        for c in loads:
            c.wait()

        def scatter(strip, wv, iiv, iov, nnz_pw, rowbase, half):
            @pl.loop(0, nnz_pw, step=_LANES)
            def _(j):
                r_idx = iiv[pl.ds(j, _LANES)] - rowbase
                io_c = iov[pl.ds(j, _LANES)]
                hi = io_c >= half
                c_idx = jnp.where(hi, io_c - half, io_c)
                bits = plsc.bitcast(wv[pl.ds(j, _LANES)], jnp.int32)
                bfv = lax.shift_right_logical(bits + 0x8000, 16)
                val = jnp.where(hi, lax.shift_left(bfv, 16), bfv)
                plsc.addupdate_scatter(strip, [r_idx, c_idx], val,
                                       mask=jnp.logical_not(hi))
                plsc.addupdate_scatter(strip, [r_idx, c_idx], val, mask=hi)

        scatter(strip0, wv0, ii0v, io0v, nnz0_pw, wid * rows0_pw, hdim // 2)
        out0 = pltpu.async_copy(
            strip0, out0_hbm.at[pl.ds(wid * rows0_pw, rows0_pw)], sem_o0)
        scatter(strip1, wv1, ii1v, io1v, nnz1_pw, wid * rows1_pw, dout // 2)
        out1 = pltpu.async_copy(
            strip1, out1_hbm.at[pl.ds(wid * rows1_pw, rows1_pw)], sem_o1)
        out0.wait()
        out1.wait()

    return sc_densify(w0, ii0, io0, w1, ii1, io1)


def _mlp(x, w0p, b0, w1p, b1):
    batch, din = x.shape
    hdim = 2 * w0p.shape[1]
    dout = 2 * w1p.shape[1]
    bb = 128

    def unpack(dst, packed):
        half = packed.shape[1]
        lo = jax.lax.bitcast_convert_type(
            jax.lax.shift_left(packed, 16), jnp.float32)
        hi = jax.lax.bitcast_convert_type(packed & jnp.int32(-65536), jnp.float32)
        dst[:, :half] = lo.astype(jnp.bfloat16)
        dst[:, half:] = hi.astype(jnp.bfloat16)

    def body(x_ref, w0_ref, b0_ref, w1_ref, b1_ref, o_ref, w0b, w1b):
        @pl.when(pl.program_id(0) == 0)
        def _():
            unpack(w0b, w0_ref[...])
            unpack(w1b, w1_ref[...])

        h = (
            jnp.dot(
                x_ref[...].astype(jnp.bfloat16),
                w0b[...],
                preferred_element_type=jnp.float32,
            )
            + b0_ref[...][None, :]
        )
        o_ref[...] = (
            jnp.dot(
                h.astype(jnp.bfloat16),
                w1b[...],
                preferred_element_type=jnp.float32,
            )
            + b1_ref[...][None, :]
        )

    return pl.pallas_call(
        body,
        grid=(batch // bb,),
        in_specs=[
            pl.BlockSpec((bb, din), lambda i: (i, 0)),
            pl.BlockSpec((din, hdim // 2), lambda i: (0, 0)),
            pl.BlockSpec((hdim,), lambda i: (0,)),
            pl.BlockSpec((hdim, dout // 2), lambda i: (0, 0)),
            pl.BlockSpec((dout,), lambda i: (0,)),
        ],
        out_specs=pl.BlockSpec((bb, dout), lambda i: (i, 0)),
        out_shape=jax.ShapeDtypeStruct((batch, dout), jnp.float32),
        scratch_shapes=[
            pltpu.VMEM((din, hdim), jnp.bfloat16),
            pltpu.VMEM((hdim, dout), jnp.bfloat16),
        ],
    )(x, w0p, b0, w1p, b1)


def kernel(x, w0, b0, w1, b1, ind_in0, ind_out0, ind_in1, ind_out1):
    din = x.shape[1]
    hdim = b0.shape[0]
    dout = b1.shape[0]
    w0d, w1d = _densify2(w0, ind_in0, ind_out0, w1, ind_in1, ind_out1,
                         din, hdim, dout)
    return _mlp(x, w0d, b0, w1d, b1)


# speedup vs baseline: 1.0667x; 1.0667x over previous
import functools

import jax
import jax.numpy as jnp
from jax import lax
from jax.experimental import pallas as pl
from jax.experimental.pallas import tpu as pltpu
from jax.experimental.pallas import tpu_sc as plsc

_LANES = 16
_NUM_CORES = 2
_NUM_SUBCORES = 16
_NW = _NUM_CORES * _NUM_SUBCORES


def _densify2(w0, ii0, io0, w1, ii1, io1, din, hdim, dout):
    nnz0, nnz1 = w0.shape[0], w1.shape[0]
    max_nnz_pw = max(nnz0, nnz1) // _NW
    max_words_pw = max(din * hdim, hdim * dout) // _NW

    mesh = plsc.VectorSubcoreMesh(core_axis_name="c", subcore_axis_name="s")

    nnz0_pw, nnz1_pw = nnz0 // _NW, nnz1 // _NW
    rows0_pw, rows1_pw = din // _NW, hdim // _NW

    @functools.partial(
        pl.kernel,
        out_type=(
            jax.ShapeDtypeStruct((din, hdim // 2), jnp.int32),
            jax.ShapeDtypeStruct((hdim, dout // 2), jnp.int32),
        ),
        mesh=mesh,
        compiler_params=pltpu.CompilerParams(needs_layout_passes=False),
        scratch_types=[
            pltpu.VMEM((rows0_pw, hdim // 2), jnp.int32),
            pltpu.VMEM((rows1_pw, dout // 2), jnp.int32),
            pltpu.VMEM((nnz0_pw,), jnp.float32),
            pltpu.VMEM((nnz0_pw,), jnp.int32),
            pltpu.VMEM((nnz0_pw,), jnp.int32),
            pltpu.VMEM((nnz1_pw,), jnp.float32),
            pltpu.VMEM((nnz1_pw,), jnp.int32),
            pltpu.VMEM((nnz1_pw,), jnp.int32),
            pltpu.SemaphoreType.DMA,
            pltpu.SemaphoreType.DMA,
            pltpu.SemaphoreType.DMA,
        ],
    )
    def sc_densify(w0_hbm, ii0_hbm, io0_hbm, w1_hbm, ii1_hbm, io1_hbm,
                   out0_hbm, out1_hbm, strip0, strip1,
                   wv0, ii0v, io0v, wv1, ii1v, io1v,
                   sem_in, sem_o0, sem_o1):
        wid = lax.axis_index("s") * _NUM_CORES + lax.axis_index("c")
        n0, n1 = wid * nnz0_pw, wid * nnz1_pw

        loads = [
            pltpu.async_copy(w0_hbm.at[pl.ds(n0, nnz0_pw)], wv0, sem_in),
            pltpu.async_copy(ii0_hbm.at[pl.ds(n0, nnz0_pw)], ii0v, sem_in),
            pltpu.async_copy(io0_hbm.at[pl.ds(n0, nnz0_pw)], io0v, sem_in),
            pltpu.async_copy(w1_hbm.at[pl.ds(n1, nnz1_pw)], wv1, sem_in),
            pltpu.async_copy(ii1_hbm.at[pl.ds(n1, nnz1_pw)], ii1v, sem_in),
            pltpu.async_copy(io1_hbm.at[pl.ds(n1, nnz1_pw)], io1v, sem_in),
        ]

        zeros = jnp.zeros((_LANES,), jnp.int32)

        def zero_strip(strip, rows, ncols_w):
            @pl.loop(0, rows)
            def _(r):
                @pl.loop(0, ncols_w, step=_LANES * 8)
                def _(c):
                    for u in range(8):
                        strip[r, pl.ds(c + u * _LANES, _LANES)] = zeros

        zero_strip(strip0, rows0_pw, hdim // 2)
        zero_strip(strip1, rows1_pw, dout // 2)
        for c in loads:
            c.wait()

        def scatter(strip, wv, iiv, iov, nnz_pw, rowbase, half):
            @pl.loop(0, nnz_pw, step=_LANES)
            def _(j):
                r_idx = iiv[pl.ds(j, _LANES)] - rowbase
                io_c = iov[pl.ds(j, _LANES)]
                hi = io_c >= half
                c_idx = jnp.where(hi, io_c - half, io_c)
                bits = plsc.bitcast(wv[pl.ds(j, _LANES)], jnp.int32)
                bfv = lax.shift_right_logical(bits + 0x8000, 16)
                val = jnp.where(hi, lax.shift_left(bfv, 16), bfv)
                plsc.addupdate_scatter(strip, [r_idx, c_idx], val,
                                       mask=jnp.logical_not(hi))
                plsc.addupdate_scatter(strip, [r_idx, c_idx], val, mask=hi)

        scatter(strip0, wv0, ii0v, io0v, nnz0_pw, wid * rows0_pw, hdim // 2)
        out0 = pltpu.async_copy(
            strip0, out0_hbm.at[pl.ds(wid * rows0_pw, rows0_pw)], sem_o0)
        scatter(strip1, wv1, ii1v, io1v, nnz1_pw, wid * rows1_pw, dout // 2)
        out1 = pltpu.async_copy(
            strip1, out1_hbm.at[pl.ds(wid * rows1_pw, rows1_pw)], sem_o1)
        out0.wait()
        out1.wait()

    return sc_densify(w0, ii0, io0, w1, ii1, io1)


def _mlp(x, w0p, b0, w1p, b1):
    batch, din = x.shape
    hdim = 2 * w0p.shape[1]
    dout = 2 * w1p.shape[1]
    bb = 512

    def unpack(dst, packed):
        half = packed.shape[1]
        lo = jax.lax.bitcast_convert_type(
            jax.lax.shift_left(packed, 16), jnp.float32)
        hi = jax.lax.bitcast_convert_type(packed & jnp.int32(-65536), jnp.float32)
        dst[:, :half] = lo.astype(jnp.bfloat16)
        dst[:, half:] = hi.astype(jnp.bfloat16)

    def body(x_ref, w0_ref, b0_ref, w1_ref, b1_ref, o_ref, w0b, w1b):
        @pl.when(pl.program_id(0) == 0)
        def _():
            unpack(w0b, w0_ref[...])
            unpack(w1b, w1_ref[...])

        h = (
            jnp.dot(
                x_ref[...].astype(jnp.bfloat16),
                w0b[...],
                preferred_element_type=jnp.float32,
            )
            + b0_ref[...][None, :]
        )
        o_ref[...] = (
            jnp.dot(
                h.astype(jnp.bfloat16),
                w1b[...],
                preferred_element_type=jnp.float32,
            )
            + b1_ref[...][None, :]
        )

    return pl.pallas_call(
        body,
        grid=(batch // bb,),
        in_specs=[
            pl.BlockSpec((bb, din), lambda i: (i, 0)),
            pl.BlockSpec((din, hdim // 2), lambda i: (0, 0)),
            pl.BlockSpec((hdim,), lambda i: (0,)),
            pl.BlockSpec((hdim, dout // 2), lambda i: (0, 0)),
            pl.BlockSpec((dout,), lambda i: (0,)),
        ],
        out_specs=pl.BlockSpec((bb, dout), lambda i: (i, 0)),
        out_shape=jax.ShapeDtypeStruct((batch, dout), jnp.float32),
        scratch_shapes=[
            pltpu.VMEM((din, hdim), jnp.bfloat16),
            pltpu.VMEM((hdim, dout), jnp.bfloat16),
        ],
    )(x, w0p, b0, w1p, b1)


def kernel(x, w0, b0, w1, b1, ind_in0, ind_out0, ind_in1, ind_out1):
    din = x.shape[1]
    hdim = b0.shape[0]
    dout = b1.shape[0]
    w0d, w1d = _densify2(w0, ind_in0, ind_out0, w1, ind_in1, ind_out1,
                         din, hdim, dout)
    return _mlp(x, w0d, b0, w1d, b1)
